# baseline (speedup 1.0000x reference)
import functools

import jax
import jax.numpy as jnp
from jax import lax
from jax.experimental import pallas as pl
from jax.experimental.pallas import tpu as pltpu
from jax.experimental.pallas import tpu_sc as plsc

_VOCAB = 1000000
_MOD = _VOCAB - 1
_B, _S = 4, 4096
_N = _B * _S
_D = 128
_M = 1024

_NC, _NS = 2, 16
_NW = _NC * _NS
_CH = _S // _NW


def _mod999999(x):
    m = jnp.int32(_MOD)
    k = jnp.int32(48577)
    msk = jnp.int32(0xFFFFF)
    for _ in range(3):
        x = (x >> 20) * k + (x & msk)
    x = jnp.where(x >= m, x - m, x)
    x = jnp.where(x < 0, x + m, x)
    return x


_NT = 16
_CT = _S // _NT
_NCHUNK = _CT // _CH


def _hash_chunk(base, tok_v, tok_off, idx_bi_v, idx_tri_v):
    for j in range(_CH // 16):
        off = j * 16
        t0 = tok_v[pl.ds(tok_off + off, 16)]
        tm1 = tok_v[pl.ds(tok_off - 1 + off, 16)]
        tm2 = tok_v[pl.ds(tok_off - 2 + off, 16)]
        a = t0 * jnp.int32(36313)
        b = tm1 * jnp.int32(27191)
        g = tm2 * jnp.int32(51497)
        hb = _mod999999(a ^ b)
        ht = _mod999999(a ^ b ^ g)
        col = (base + off + lax.iota(jnp.int32, 16)) & jnp.int32(_S - 1)
        hb = jnp.where(col == 0, jnp.int32(_MOD), hb)
        ht = jnp.where(col <= 1, jnp.int32(_MOD), ht)
        idx_bi_v[pl.ds(off, 16)] = hb
        idx_tri_v[pl.ds(off, 16)] = ht


def _sc_body(slab, tok_hbm, table_hbm, h_hbm,
             tok_v, idx_bi0, idx_tri0, idx_bi1, idx_tri1,
             rows_bi0, rows_tri0, rows_bi1, rows_tri1,
             sem_bi, sem_tri, sem_wb_bi, sem_wb_tri):
    wid = lax.axis_index("s")
    tbase = wid * _CT
    base = slab * _S + tbase

    pltpu.sync_copy(tok_hbm.at[pl.ds(base, _CT)], tok_v.at[pl.ds(8, _CT)])

    @pl.when(base > 0)
    def _():
        pltpu.sync_copy(tok_hbm.at[pl.ds(base - 8, 8)], tok_v.at[pl.ds(0, 8)])

    idx_bi = [idx_bi0, idx_bi1]
    idx_tri = [idx_tri0, idx_tri1]
    rows_bi = [rows_bi0, rows_bi1]
    rows_tri = [rows_tri0, rows_tri1]
    gathers = [None] * _NCHUNK
    wbs = [None] * _NCHUNK

    _hash_chunk(base, tok_v, 8, idx_bi[0], idx_tri[0])
    gathers[0] = (pltpu.async_copy(table_hbm.at[idx_bi[0]], rows_bi[0], sem_bi),
                  pltpu.async_copy(table_hbm.at[idx_tri[0]], rows_tri[0], sem_tri))
    for ch in range(_NCHUNK):
        sl = ch % 2
        nxt = ch + 1
        if nxt < _NCHUNK:
            if nxt >= 2:
                wbs[nxt - 2][0].wait()
                wbs[nxt - 2][1].wait()
            nsl = nxt % 2
            _hash_chunk(base + nxt * _CH, tok_v, 8 + nxt * _CH,
                        idx_bi[nsl], idx_tri[nsl])
            gathers[nxt] = (
                pltpu.async_copy(table_hbm.at[idx_bi[nsl]], rows_bi[nsl], sem_bi),
                pltpu.async_copy(table_hbm.at[idx_tri[nsl]], rows_tri[nsl], sem_tri))
        gathers[ch][0].wait()
        gathers[ch][1].wait()
        dst = pl.ds(tbase + ch * _CH, _CH)
        wbs[ch] = (pltpu.async_copy(rows_bi[sl], h_hbm.at[0, dst], sem_wb_bi),
                   pltpu.async_copy(rows_tri[sl], h_hbm.at[1, dst], sem_wb_tri))
    for ch in range(max(0, _NCHUNK - 2), _NCHUNK):
        wbs[ch][0].wait()
        wbs[ch][1].wait()


def _make_sc(slab):
    return pl.kernel(
        functools.partial(_sc_body, slab),
        mesh=plsc.VectorSubcoreMesh(core_axis_name="c", subcore_axis_name="s",
                                    num_cores=1),
        out_type=jax.ShapeDtypeStruct((2, _S, _D), jnp.float32),
        scratch_types=[
            pltpu.VMEM((_CT + 8,), jnp.int32),
            pltpu.VMEM((_CH,), jnp.int32),
            pltpu.VMEM((_CH,), jnp.int32),
            pltpu.VMEM((_CH,), jnp.int32),
            pltpu.VMEM((_CH,), jnp.int32),
            pltpu.VMEM((_CH, _D), jnp.float32),
            pltpu.VMEM((_CH, _D), jnp.float32),
            pltpu.VMEM((_CH, _D), jnp.float32),
            pltpu.VMEM((_CH, _D), jnp.float32),
            pltpu.SemaphoreType.DMA,
            pltpu.SemaphoreType.DMA,
            pltpu.SemaphoreType.DMA,
            pltpu.SemaphoreType.DMA,
        ],
    )


_sc_gathers = [_make_sc(k) for k in range(_B)]

_BM = 256
_SLAB_BLOCKS = _S // _BM


def _mm_first_body(scale_ref, h_ref, w_ref, o_ref):
    h = h_ref[0] + h_ref[1]
    acc = lax.dot_general(h, w_ref[...],
                          (((1,), (1,)), ((), ())),
                          preferred_element_type=jnp.float32)
    o_ref[...] = acc * scale_ref[0]


def _mm_chain_body(ob_ref, scale_ref, h_ref, w_ref, o_ref):
    del ob_ref
    _mm_first_body(scale_ref, h_ref, w_ref, o_ref)


def _matmul_slab(k, out_buf, h, w, scale):
    if k == 0:
        return pl.pallas_call(
            _mm_first_body,
            grid=(_SLAB_BLOCKS,),
            in_specs=[
                pl.BlockSpec(memory_space=pltpu.SMEM),
                pl.BlockSpec((2, _BM, _D), lambda i: (0, i, 0)),
                pl.BlockSpec((_M, _D), lambda i: (0, 0)),
            ],
            out_specs=pl.BlockSpec((_BM, _M), lambda i: (i, 0)),
            out_shape=jax.ShapeDtypeStruct((_N, _M), jnp.float32),
        )(scale, h, w)
    return pl.pallas_call(
        _mm_chain_body,
        grid=(_SLAB_BLOCKS,),
        in_specs=[
            pl.BlockSpec(memory_space=pl.ANY),
            pl.BlockSpec(memory_space=pltpu.SMEM),
            pl.BlockSpec((2, _BM, _D), lambda i: (0, i, 0)),
            pl.BlockSpec((_M, _D), lambda i: (0, 0)),
        ],
        out_specs=pl.BlockSpec((_BM, _M), lambda i, k=k: (i + k * _SLAB_BLOCKS, 0)),
        out_shape=jax.ShapeDtypeStruct((_N, _M), jnp.float32),
        input_output_aliases={0: 0},
    )(out_buf, scale, h, w)


def kernel(token_ids, embed_table, proj_w, scale):
    tok = token_ids.reshape(_N)
    scale1 = scale.astype(jnp.float32).reshape(1)
    hs = [_sc_gathers[k](tok, embed_table) for k in range(_B)]
    out = None
    for k in range(_B):
        out = _matmul_slab(k, out, hs[k], proj_w, scale1)
    return out.reshape(_B, _S, _M)


# speedup vs baseline: 1.2903x; 1.2903x over previous
import functools

import jax
import jax.numpy as jnp
from jax import lax
from jax.experimental import pallas as pl
from jax.experimental.pallas import tpu as pltpu
from jax.experimental.pallas import tpu_sc as plsc

_VOCAB = 1000000
_MOD = _VOCAB - 1
_B, _S = 4, 4096
_N = _B * _S
_D = 128
_M = 1024

_NT = 16
_CT = _N // _NT
_CH = 128
_NCHUNK = _CT // _CH


def _mod999999(x):
    m = jnp.int32(_MOD)
    k = jnp.int32(48577)
    msk = jnp.int32(0xFFFFF)
    for _ in range(3):
        x = (x >> 20) * k + (x & msk)
    x = jnp.where(x >= m, x - m, x)
    x = jnp.where(x < 0, x + m, x)
    return x


def _hash_chunk(base, tok_v, tok_off, idx_bi_v, idx_tri_v):
    for j in range(_CH // 16):
        off = j * 16
        t0 = tok_v[pl.ds(tok_off + off, 16)]
        tm1 = tok_v[pl.ds(tok_off - 1 + off, 16)]
        tm2 = tok_v[pl.ds(tok_off - 2 + off, 16)]
        a = t0 * jnp.int32(36313)
        b = tm1 * jnp.int32(27191)
        g = tm2 * jnp.int32(51497)
        hb = _mod999999(a ^ b)
        ht = _mod999999(a ^ b ^ g)
        col = (base + off + lax.iota(jnp.int32, 16)) & jnp.int32(_S - 1)
        hb = jnp.where(col == 0, jnp.int32(_MOD), hb)
        ht = jnp.where(col <= 1, jnp.int32(_MOD), ht)
        idx_bi_v[pl.ds(off, 16)] = hb
        idx_tri_v[pl.ds(off, 16)] = ht


def _sc_body(tok_hbm, table_hbm, h_hbm,
             tok_v, idx_bi0, idx_tri0, idx_bi1, idx_tri1,
             rows_bi0, rows_tri0, rows_bi1, rows_tri1,
             sem_bi, sem_tri, sem_wb_bi, sem_wb_tri):
    wid = lax.axis_index("s")
    base = wid * _CT

    pltpu.sync_copy(tok_hbm.at[pl.ds(base, _CT)], tok_v.at[pl.ds(8, _CT)])

    @pl.when(base > 0)
    def _():
        pltpu.sync_copy(tok_hbm.at[pl.ds(base - 8, 8)], tok_v.at[pl.ds(0, 8)])

    idx_bi = [idx_bi0, idx_bi1]
    idx_tri = [idx_tri0, idx_tri1]
    rows_bi = [rows_bi0, rows_bi1]
    rows_tri = [rows_tri0, rows_tri1]
    gathers = [None] * _NCHUNK
    wbs = [None] * _NCHUNK

    _hash_chunk(base, tok_v, 8, idx_bi[0], idx_tri[0])
    gathers[0] = (pltpu.async_copy(table_hbm.at[idx_bi[0]], rows_bi[0], sem_bi),
                  pltpu.async_copy(table_hbm.at[idx_tri[0]], rows_tri[0], sem_tri))
    for ch in range(_NCHUNK):
        sl = ch % 2
        nxt = ch + 1
        if nxt < _NCHUNK:
            if nxt >= 2:
                wbs[nxt - 2][0].wait()
                wbs[nxt - 2][1].wait()
            nsl = nxt % 2
            _hash_chunk(base + nxt * _CH, tok_v, 8 + nxt * _CH,
                        idx_bi[nsl], idx_tri[nsl])
            gathers[nxt] = (
                pltpu.async_copy(table_hbm.at[idx_bi[nsl]], rows_bi[nsl], sem_bi),
                pltpu.async_copy(table_hbm.at[idx_tri[nsl]], rows_tri[nsl], sem_tri))
        gathers[ch][0].wait()
        gathers[ch][1].wait()
        dst = pl.ds(base + ch * _CH, _CH)
        wbs[ch] = (pltpu.async_copy(rows_bi[sl], h_hbm.at[0, dst], sem_wb_bi),
                   pltpu.async_copy(rows_tri[sl], h_hbm.at[1, dst], sem_wb_tri))
    for ch in range(max(0, _NCHUNK - 2), _NCHUNK):
        wbs[ch][0].wait()
        wbs[ch][1].wait()


_sc_gather = pl.kernel(
    _sc_body,
    mesh=plsc.VectorSubcoreMesh(core_axis_name="c", subcore_axis_name="s",
                                num_cores=1),
    out_type=jax.ShapeDtypeStruct((2, _N, _D), jnp.float32),
    scratch_types=[
        pltpu.VMEM((_CT + 8,), jnp.int32),
        pltpu.VMEM((_CH,), jnp.int32),
        pltpu.VMEM((_CH,), jnp.int32),
        pltpu.VMEM((_CH,), jnp.int32),
        pltpu.VMEM((_CH,), jnp.int32),
        pltpu.VMEM((_CH, _D), jnp.float32),
        pltpu.VMEM((_CH, _D), jnp.float32),
        pltpu.VMEM((_CH, _D), jnp.float32),
        pltpu.VMEM((_CH, _D), jnp.float32),
        pltpu.SemaphoreType.DMA,
        pltpu.SemaphoreType.DMA,
        pltpu.SemaphoreType.DMA,
        pltpu.SemaphoreType.DMA,
    ],
)

_BM = 1024


def _mm_body(scale_ref, h_ref, w_ref, o_ref):
    h = h_ref[0] + h_ref[1]
    acc = lax.dot_general(h, w_ref[...],
                          (((1,), (1,)), ((), ())),
                          preferred_element_type=jnp.float32)
    o_ref[...] = acc * scale_ref[0]


def _matmul(h2, w, scale):
    return pl.pallas_call(
        _mm_body,
        grid=(_N // _BM,),
        in_specs=[
            pl.BlockSpec(memory_space=pltpu.SMEM),
            pl.BlockSpec((2, _BM, _D), lambda i: (0, i, 0)),
            pl.BlockSpec((_M, _D), lambda i: (0, 0)),
        ],
        out_specs=pl.BlockSpec((_BM, _M), lambda i: (i, 0)),
        out_shape=jax.ShapeDtypeStruct((_N, _M), jnp.float32),
    )(scale, h2, w)


def kernel(token_ids, embed_table, proj_w, scale):
    tok = token_ids.reshape(_N)
    scale1 = scale.astype(jnp.float32).reshape(1)
    h2 = _sc_gather(tok, embed_table)
    out = _matmul(h2, proj_w, scale1)
    return out.reshape(_B, _S, _M)


# speedup vs baseline: 1.3481x; 1.0448x over previous
import functools

import jax
import jax.numpy as jnp
from jax import lax
from jax.experimental import pallas as pl
from jax.experimental.pallas import tpu as pltpu
from jax.experimental.pallas import tpu_sc as plsc

_VOCAB = 1000000
_MOD = _VOCAB - 1
_B, _S = 4, 4096
_N = _B * _S
_D = 128
_M = 1024

_NT = 16
_CT = _N // _NT
_CH = 128
_NCHUNK = _CT // _CH


def _mod999999(x):
    m = jnp.int32(_MOD)
    k = jnp.int32(48577)
    msk = jnp.int32(0xFFFFF)
    for _ in range(3):
        x = (x >> 20) * k + (x & msk)
    x = jnp.where(x >= m, x - m, x)
    x = jnp.where(x < 0, x + m, x)
    return x


def _hash_chunk(base, tok_v, tok_off, idx_bi_v, idx_tri_v):
    for j in range(_CH // 16):
        off = j * 16
        t0 = tok_v[pl.ds(tok_off + off, 16)]
        tm1 = tok_v[pl.ds(tok_off - 1 + off, 16)]
        tm2 = tok_v[pl.ds(tok_off - 2 + off, 16)]
        a = t0 * jnp.int32(36313)
        b = tm1 * jnp.int32(27191)
        g = tm2 * jnp.int32(51497)
        hb = _mod999999(a ^ b)
        ht = _mod999999(a ^ b ^ g)
        col = (base + off + lax.iota(jnp.int32, 16)) & jnp.int32(_S - 1)
        hb = jnp.where(col == 0, jnp.int32(_MOD), hb)
        ht = jnp.where(col <= 1, jnp.int32(_MOD), ht)
        idx_bi_v[pl.ds(off, 16)] = hb
        idx_tri_v[pl.ds(off, 16)] = ht


def _sc_body(tok_hbm, table_hbm, h_hbm,
             tok_v, idx_bi0, idx_tri0, idx_bi1, idx_tri1,
             rows_bi0, rows_tri0, rows_bi1, rows_tri1,
             sem_bi, sem_tri, sem_wb_bi, sem_wb_tri):
    wid = lax.axis_index("s")
    base = wid * _CT

    pltpu.sync_copy(tok_hbm.at[pl.ds(base, _CT)], tok_v.at[pl.ds(8, _CT)])

    @pl.when(base > 0)
    def _():
        pltpu.sync_copy(tok_hbm.at[pl.ds(base - 8, 8)], tok_v.at[pl.ds(0, 8)])

    idx_bi = [idx_bi0, idx_bi1]
    idx_tri = [idx_tri0, idx_tri1]
    rows_bi = [rows_bi0, rows_bi1]
    rows_tri = [rows_tri0, rows_tri1]
    gathers = [None] * _NCHUNK
    wbs = [None] * _NCHUNK

    _hash_chunk(base, tok_v, 8, idx_bi[0], idx_tri[0])
    gathers[0] = (pltpu.async_copy(table_hbm.at[idx_bi[0]], rows_bi[0], sem_bi),
                  pltpu.async_copy(table_hbm.at[idx_tri[0]], rows_tri[0], sem_tri))
    for ch in range(_NCHUNK):
        sl = ch % 2
        nxt = ch + 1
        if nxt < _NCHUNK:
            if nxt >= 2:
                wbs[nxt - 2][0].wait()
                wbs[nxt - 2][1].wait()
            nsl = nxt % 2
            _hash_chunk(base + nxt * _CH, tok_v, 8 + nxt * _CH,
                        idx_bi[nsl], idx_tri[nsl])
            gathers[nxt] = (
                pltpu.async_copy(table_hbm.at[idx_bi[nsl]], rows_bi[nsl], sem_bi),
                pltpu.async_copy(table_hbm.at[idx_tri[nsl]], rows_tri[nsl], sem_tri))
        gathers[ch][0].wait()
        gathers[ch][1].wait()
        dst = pl.ds(base + ch * _CH, _CH)
        wbs[ch] = (pltpu.async_copy(rows_bi[sl], h_hbm.at[0, dst], sem_wb_bi),
                   pltpu.async_copy(rows_tri[sl], h_hbm.at[1, dst], sem_wb_tri))
    for ch in range(max(0, _NCHUNK - 2), _NCHUNK):
        wbs[ch][0].wait()
        wbs[ch][1].wait()


_sc_gather = pl.kernel(
    _sc_body,
    mesh=plsc.VectorSubcoreMesh(core_axis_name="c", subcore_axis_name="s",
                                num_cores=1),
    out_type=jax.ShapeDtypeStruct((2, _N, _D), jnp.float32),
    scratch_types=[
        pltpu.VMEM((_CT + 8,), jnp.int32),
        pltpu.VMEM((_CH,), jnp.int32),
        pltpu.VMEM((_CH,), jnp.int32),
        pltpu.VMEM((_CH,), jnp.int32),
        pltpu.VMEM((_CH,), jnp.int32),
        pltpu.VMEM((_CH, _D), jnp.float32),
        pltpu.VMEM((_CH, _D), jnp.float32),
        pltpu.VMEM((_CH, _D), jnp.float32),
        pltpu.VMEM((_CH, _D), jnp.float32),
        pltpu.SemaphoreType.DMA,
        pltpu.SemaphoreType.DMA,
        pltpu.SemaphoreType.DMA,
        pltpu.SemaphoreType.DMA,
    ],
)

_BM = 2048


def _mm_body(scale_ref, h_ref, w_ref, o_ref):
    h = h_ref[0] + h_ref[1]
    acc = lax.dot_general(h, w_ref[...],
                          (((1,), (1,)), ((), ())),
                          preferred_element_type=jnp.float32)
    o_ref[...] = acc * scale_ref[0]


def _matmul(h2, w, scale):
    return pl.pallas_call(
        _mm_body,
        grid=(_N // _BM,),
        in_specs=[
            pl.BlockSpec(memory_space=pltpu.SMEM),
            pl.BlockSpec((2, _BM, _D), lambda i: (0, i, 0)),
            pl.BlockSpec((_M, _D), lambda i: (0, 0)),
        ],
        out_specs=pl.BlockSpec((_BM, _M), lambda i: (i, 0)),
        out_shape=jax.ShapeDtypeStruct((_N, _M), jnp.float32),
    )(scale, h2, w)


def kernel(token_ids, embed_table, proj_w, scale):
    tok = token_ids.reshape(_N)
    scale1 = scale.astype(jnp.float32).reshape(1)
    h2 = _sc_gather(tok, embed_table)
    out = _matmul(h2, proj_w, scale1)
    return out.reshape(_B, _S, _M)


# speedup vs baseline: 1.3573x; 1.0069x over previous
import functools

import jax
import jax.numpy as jnp
from jax import lax
from jax.experimental import pallas as pl
from jax.experimental.pallas import tpu as pltpu
from jax.experimental.pallas import tpu_sc as plsc

_VOCAB = 1000000
_MOD = _VOCAB - 1
_B, _S = 4, 4096
_N = _B * _S
_D = 128
_M = 1024

_NT = 16
_CT = _N // _NT
_CH = 128
_NCHUNK = _CT // _CH


def _mod999999(x):
    m = jnp.int32(_MOD)
    k = jnp.int32(48577)
    msk = jnp.int32(0xFFFFF)
    for _ in range(3):
        x = (x >> 20) * k + (x & msk)
    x = jnp.where(x >= m, x - m, x)
    x = jnp.where(x < 0, x + m, x)
    return x


def _hash_chunk(base, tok_v, tok_off, idx_bi_v, idx_tri_v):
    for j in range(_CH // 16):
        off = j * 16
        t0 = tok_v[pl.ds(tok_off + off, 16)]
        tm1 = tok_v[pl.ds(tok_off - 1 + off, 16)]
        tm2 = tok_v[pl.ds(tok_off - 2 + off, 16)]
        a = t0 * jnp.int32(36313)
        b = tm1 * jnp.int32(27191)
        g = tm2 * jnp.int32(51497)
        hb = _mod999999(a ^ b)
        ht = _mod999999(a ^ b ^ g)
        col = (base + off + lax.iota(jnp.int32, 16)) & jnp.int32(_S - 1)
        hb = jnp.where(col == 0, jnp.int32(_MOD), hb)
        ht = jnp.where(col <= 1, jnp.int32(_MOD), ht)
        idx_bi_v[pl.ds(off, 16)] = hb
        idx_tri_v[pl.ds(off, 16)] = ht


def _sc_body(tok_hbm, table_hbm, h_hbm,
             tok_v, idx_bi0, idx_tri0, idx_bi1, idx_tri1,
             rows_bi0, rows_tri0, rows_bi1, rows_tri1,
             sem_bi, sem_tri, sem_wb_bi, sem_wb_tri):
    wid = lax.axis_index("s")
    base = wid * _CT

    pltpu.sync_copy(tok_hbm.at[pl.ds(base, _CT)], tok_v.at[pl.ds(8, _CT)])

    @pl.when(base > 0)
    def _():
        pltpu.sync_copy(tok_hbm.at[pl.ds(base - 8, 8)], tok_v.at[pl.ds(0, 8)])

    idx_bi = [idx_bi0, idx_bi1]
    idx_tri = [idx_tri0, idx_tri1]
    rows_bi = [rows_bi0, rows_bi1]
    rows_tri = [rows_tri0, rows_tri1]
    gathers = [None] * _NCHUNK
    wbs = [None] * _NCHUNK

    _hash_chunk(base, tok_v, 8, idx_bi[0], idx_tri[0])
    gathers[0] = (pltpu.async_copy(table_hbm.at[idx_bi[0]], rows_bi[0], sem_bi),
                  pltpu.async_copy(table_hbm.at[idx_tri[0]], rows_tri[0], sem_tri))
    for ch in range(_NCHUNK):
        sl = ch % 2
        nxt = ch + 1
        if nxt < _NCHUNK:
            if nxt >= 2:
                wbs[nxt - 2][0].wait()
                wbs[nxt - 2][1].wait()
            nsl = nxt % 2
            _hash_chunk(base + nxt * _CH, tok_v, 8 + nxt * _CH,
                        idx_bi[nsl], idx_tri[nsl])
            gathers[nxt] = (
                pltpu.async_copy(table_hbm.at[idx_bi[nsl]], rows_bi[nsl], sem_bi),
                pltpu.async_copy(table_hbm.at[idx_tri[nsl]], rows_tri[nsl], sem_tri))
        gathers[ch][0].wait()
        gathers[ch][1].wait()
        dst = pl.ds(base + ch * _CH, _CH)
        wbs[ch] = (pltpu.async_copy(rows_bi[sl], h_hbm.at[0, dst], sem_wb_bi),
                   pltpu.async_copy(rows_tri[sl], h_hbm.at[1, dst], sem_wb_tri))
    for ch in range(max(0, _NCHUNK - 2), _NCHUNK):
        wbs[ch][0].wait()
        wbs[ch][1].wait()


_sc_gather = pl.kernel(
    _sc_body,
    mesh=plsc.VectorSubcoreMesh(core_axis_name="c", subcore_axis_name="s",
                                num_cores=1),
    out_type=jax.ShapeDtypeStruct((2, _N, _D), jnp.float32),
    scratch_types=[
        pltpu.VMEM((_CT + 8,), jnp.int32),
        pltpu.VMEM((_CH,), jnp.int32),
        pltpu.VMEM((_CH,), jnp.int32),
        pltpu.VMEM((_CH,), jnp.int32),
        pltpu.VMEM((_CH,), jnp.int32),
        pltpu.VMEM((_CH, _D), jnp.float32),
        pltpu.VMEM((_CH, _D), jnp.float32),
        pltpu.VMEM((_CH, _D), jnp.float32),
        pltpu.VMEM((_CH, _D), jnp.float32),
        pltpu.SemaphoreType.DMA,
        pltpu.SemaphoreType.DMA,
        pltpu.SemaphoreType.DMA,
        pltpu.SemaphoreType.DMA,
    ],
)

_BM = 4096


def _mm_body(scale_ref, h_ref, w_ref, o_ref):
    h = h_ref[0] + h_ref[1]
    acc = lax.dot_general(h, w_ref[...],
                          (((1,), (1,)), ((), ())),
                          preferred_element_type=jnp.float32)
    o_ref[...] = acc * scale_ref[0]


def _matmul(h2, w, scale):
    return pl.pallas_call(
        _mm_body,
        grid=(_N // _BM,),
        in_specs=[
            pl.BlockSpec(memory_space=pltpu.SMEM),
            pl.BlockSpec((2, _BM, _D), lambda i: (0, i, 0)),
            pl.BlockSpec((_M, _D), lambda i: (0, 0)),
        ],
        out_specs=pl.BlockSpec((_BM, _M), lambda i: (i, 0)),
        out_shape=jax.ShapeDtypeStruct((_N, _M), jnp.float32),
    )(scale, h2, w)


def kernel(token_ids, embed_table, proj_w, scale):
    tok = token_ids.reshape(_N)
    scale1 = scale.astype(jnp.float32).reshape(1)
    h2 = _sc_gather(tok, embed_table)
    out = _matmul(h2, proj_w, scale1)
    return out.reshape(_B, _S, _M)
